# baseline (speedup 1.0000x reference)
import functools

import jax
import jax.numpy as jnp
from jax import lax
from jax.experimental import pallas as pl
from jax.experimental.pallas import tpu as pltpu
from jax.experimental.pallas import tpu_sc as plsc

N_NODES = 50000
N_EDGES = 800000
CH = 32
LANES = 16
NC = 2
NS = 16
NW = NC * NS
ECH = 128
NCHUNK = N_EDGES // ECH
KA = -(-NCHUNK // NW)
KB = -(-NCHUNK // NS)
RPS = N_NODES // NS
ZR = 125
HSTEP = 0.1
EPS = 0.001

_mesh = plsc.VectorSubcoreMesh(core_axis_name="c", subcore_axis_name="s")
_sc_params = pltpu.CompilerParams(use_tc_tiling_on_sc=False)


@functools.partial(
    pl.kernel,
    mesh=_mesh,
    out_type=[
        jax.ShapeDtypeStruct((N_EDGES, CH), jnp.float32),
        jax.ShapeDtypeStruct((N_EDGES, CH), jnp.float32),
        jax.ShapeDtypeStruct((NW, 8, LANES), jnp.float32),
    ],
    scratch_types=[
        pltpu.VMEM((KA * ECH,), jnp.int32),
        pltpu.VMEM((KA * ECH,), jnp.int32),
        pltpu.VMEM((2, ECH, 2 * CH), jnp.float32),
        pltpu.VMEM((2, ECH, 2 * CH), jnp.float32),
        pltpu.VMEM((2, ECH, CH), jnp.float32),
        pltpu.VMEM((2, ECH, CH), jnp.float32),
        pltpu.VMEM((8, LANES), jnp.float32),
        pltpu.SemaphoreType.DMA,
        pltpu.SemaphoreType.DMA,
        pltpu.SemaphoreType.DMA,
        pltpu.SemaphoreType.DMA,
    ],
    compiler_params=_sc_params,
)
def _sc_pass_a(yuv, ei, aU, aV, stats, idx_i, idx_j, bYi, bYj, wU, wV,
               stats_v, sem0, sem1, wsem0, wsem1):
    cid = lax.axis_index("c")
    sid = lax.axis_index("s")
    wid = sid * NC + cid
    extra = NCHUNK - (KA - 1) * NW
    nk = jnp.where(wid < extra, KA, KA - 1)
    cb = jnp.where(wid < extra, wid * KA, extra + wid * (KA - 1))
    sems = (sem0, sem1)
    wsems = (wsem0, wsem1)
    zero = jnp.zeros((LANES,), jnp.float32)
    for r in range(8):
        stats_v[r, :] = zero

    e0 = cb * ECH
    pltpu.sync_copy(ei.at[0, pl.ds(e0, (KA - 1) * ECH)],
                    idx_i.at[pl.ds(0, (KA - 1) * ECH)])
    pltpu.sync_copy(ei.at[1, pl.ds(e0, (KA - 1) * ECH)],
                    idx_j.at[pl.ds(0, (KA - 1) * ECH)])

    @pl.when(nk == KA)
    def _():
        tail = e0 + (KA - 1) * ECH
        pltpu.sync_copy(ei.at[0, pl.ds(tail, ECH)],
                        idx_i.at[pl.ds((KA - 1) * ECH, ECH)])
        pltpu.sync_copy(ei.at[1, pl.ds(tail, ECH)],
                        idx_j.at[pl.ds((KA - 1) * ECH, ECH)])

    def issue(s, k):
        pltpu.async_copy(yuv.at[idx_i.at[pl.ds(k * ECH, ECH)]], bYi.at[s],
                         sems[s])
        pltpu.async_copy(yuv.at[idx_j.at[pl.ds(k * ECH, ECH)]], bYj.at[s],
                         sems[s])

    def drain_write(s):
        pltpu.make_async_copy(aU.at[pl.ds(0, ECH)], wU.at[s],
                              wsems[s]).wait()
        pltpu.make_async_copy(aU.at[pl.ds(0, ECH)], wV.at[s],
                              wsems[s]).wait()

    def consume(s, k):
        for dst in (bYi.at[s], bYj.at[s]):
            pltpu.make_async_copy(yuv.at[pl.ds(0, ECH)], dst, sems[s]).wait()

        @pl.when(k >= 2)
        def _():
            drain_write(s)

        def row_body(r2, st):
            su0, su1, qu0, qu1, sv0, sv1, qv0, qv1 = st
            for rr in range(4):
                r = 4 * r2 + rr
                s0 = pl.ds(0, LANES)
                s1 = pl.ds(LANES, LANES)
                s2 = pl.ds(2 * LANES, LANES)
                s3 = pl.ds(3 * LANES, LANES)
                au0 = bYi[s, r, s0] - bYj[s, r, s0]
                au1 = bYi[s, r, s1] - bYj[s, r, s1]
                av0 = bYi[s, r, s2] - bYj[s, r, s2]
                av1 = bYi[s, r, s3] - bYj[s, r, s3]
                wU[s, r, pl.ds(0, LANES)] = au0
                wU[s, r, pl.ds(LANES, LANES)] = au1
                wV[s, r, pl.ds(0, LANES)] = av0
                wV[s, r, pl.ds(LANES, LANES)] = av1
                su0 = su0 + au0
                su1 = su1 + au1
                qu0 = qu0 + au0 * au0
                qu1 = qu1 + au1 * au1
                sv0 = sv0 + av0
                sv1 = sv1 + av1
                qv0 = qv0 + av0 * av0
                qv1 = qv1 + av1 * av1
            return (su0, su1, qu0, qu1, sv0, sv1, qv0, qv1)

        st = lax.fori_loop(0, ECH // 4, row_body, (zero,) * 8)
        for r in range(8):
            stats_v[r, :] = stats_v[r, :] + st[r]
        base = (cb + k) * ECH
        pltpu.async_copy(wU.at[s], aU.at[pl.ds(base, ECH)], wsems[s])
        pltpu.async_copy(wV.at[s], aV.at[pl.ds(base, ECH)], wsems[s])

    issue(0, 0)

    def pair_body(t, carry):
        k0 = 2 * t
        k1 = k0 + 1

        @pl.when(k1 < nk)
        def _():
            issue(1, k1)

        consume(0, k0)

        @pl.when(k0 + 2 < nk)
        def _():
            issue(0, k0 + 2)

        @pl.when(k1 < nk)
        def _():
            consume(1, k1)

        return carry

    lax.fori_loop(0, KA // 2, pair_body, 0)
    drain_write(0)
    drain_write(1)
    pltpu.sync_copy(stats_v, stats.at[wid])


@functools.partial(
    pl.kernel,
    mesh=_mesh,
    out_type=[
        jax.ShapeDtypeStruct((NC, N_NODES, CH), jnp.float32),
    ],
    scratch_types=[
        pltpu.VMEM_SHARED((N_NODES, CH), jnp.float32),
        pltpu.VMEM((2, ECH, CH), jnp.float32),
        pltpu.VMEM((2, ECH, CH), jnp.float32),
        pltpu.VMEM((2, ECH, CH), jnp.float32),
        pltpu.VMEM((2, ECH), jnp.int32),
        pltpu.VMEM((2, ECH), jnp.int32),
        pltpu.VMEM((ZR, CH), jnp.float32),
        pltpu.VMEM((8, LANES), jnp.float32),
        pltpu.SemaphoreType.DMA,
        pltpu.SemaphoreType.DMA,
        pltpu.SemaphoreType.DMA,
        pltpu.SemaphoreType.DMA,
    ],
    compiler_params=_sc_params,
)
def _sc_pass_b(aU, aV, ei, aff, S, acc, aBuf, sBuf, nBuf, idx_i, idx_j, zbuf,
               aff_v, sem0, sem1, ssem0, ssem1):
    cid = lax.axis_index("c")
    sid = lax.axis_index("s")
    sems = (sem0, sem1)
    ssems = (ssem0, ssem1)
    extra = NCHUNK - (KB - 1) * NS
    nk = jnp.where(sid < extra, KB, KB - 1)
    zero = jnp.zeros((LANES,), jnp.float32)

    def zrow(r, carry):
        zbuf[r, pl.ds(0, LANES)] = zero
        zbuf[r, pl.ds(LANES, LANES)] = zero
        return carry

    lax.fori_loop(0, ZR, zrow, 0)
    for t in range(RPS // ZR):
        pltpu.sync_copy(zbuf, acc.at[pl.ds(sid * RPS + t * ZR, ZR)])
    plsc.subcore_barrier()

    pltpu.sync_copy(aff, aff_v)
    mul0 = aff_v[4 * cid + 0, :]
    mul1 = aff_v[4 * cid + 1, :]
    sub0 = aff_v[4 * cid + 2, :]
    sub1 = aff_v[4 * cid + 3, :]
    outscale = jnp.where(cid == 0, 1.0, 0.5).astype(jnp.float32)
    sign = jnp.where(cid == 0, -1.0, 1.0).astype(jnp.float32)

    def issue(s, k):
        base = (k * NS + sid) * ECH

        @pl.when(cid == 0)
        def _():
            pltpu.async_copy(aU.at[pl.ds(base, ECH)], aBuf.at[s], sems[s])

        @pl.when(cid == 1)
        def _():
            pltpu.async_copy(aV.at[pl.ds(base, ECH)], aBuf.at[s], sems[s])

        pltpu.async_copy(ei.at[0, pl.ds(base, ECH)], idx_i.at[s], sems[s])
        pltpu.async_copy(ei.at[1, pl.ds(base, ECH)], idx_j.at[s], sems[s])

    def drain_scatter(s):
        pltpu.make_async_copy(aU.at[pl.ds(0, ECH)], sBuf.at[s],
                              ssems[s]).wait()
        pltpu.make_async_copy(aU.at[pl.ds(0, ECH)], nBuf.at[s],
                              ssems[s]).wait()

    def consume(s, k):
        pltpu.make_async_copy(aU.at[pl.ds(0, ECH)], aBuf.at[s],
                              sems[s]).wait()
        pltpu.make_async_copy(ei.at[0, pl.ds(0, ECH)], idx_i.at[s],
                              sems[s]).wait()
        pltpu.make_async_copy(ei.at[0, pl.ds(0, ECH)], idx_j.at[s],
                              sems[s]).wait()

        @pl.when(k >= 2)
        def _():
            drain_scatter(s)

        def row_body(r2, rc):
            for rr in range(2):
                r = 2 * r2 + rr
                for h, (m, sb) in enumerate(((mul0, sub0), (mul1, sub1))):
                    sl = pl.ds(h * LANES, LANES)
                    a = aBuf[s, r, sl]
                    d = jnp.maximum(a * m - sb, 0.0) * outscale
                    sBuf[s, r, sl] = d
                    nBuf[s, r, sl] = d * sign
            return rc

        lax.fori_loop(0, ECH // 2, row_body, 0)
        pltpu.async_copy(sBuf.at[s], acc.at[idx_i.at[s]], ssems[s], add=True)
        pltpu.async_copy(nBuf.at[s], acc.at[idx_j.at[s]], ssems[s], add=True)

    issue(0, 0)

    def pair_body(t, carry):
        k0 = 2 * t
        k1 = k0 + 1

        @pl.when(k1 < nk)
        def _():
            issue(1, k1)

        @pl.when(k0 < nk)
        def _():
            consume(0, k0)

        @pl.when(k0 + 2 < nk)
        def _():
            issue(0, k0 + 2)

        @pl.when(k1 < nk)
        def _():
            consume(1, k1)

        return carry

    lax.fori_loop(0, (KB + 1) // 2, pair_body, 0)
    drain_scatter(0)
    drain_scatter(1)
    plsc.subcore_barrier()
    for t in range(RPS // ZR):
        row0 = sid * RPS + t * ZR
        pltpu.sync_copy(acc.at[pl.ds(row0, ZR)], S.at[cid, pl.ds(row0, ZR)])


NB = 5000
GRID = N_NODES // NB
_DN_RR = (((1,), (1,)), ((), ()))
_DN_RC = (((1,), (0,)), ((), ()))


def _tc_pre_body(x_ref, kuv_ref, kr_ref, yuv_ref, ri_ref, rs_ref):
    xb = x_ref[...]
    yuv_ref[...] = lax.dot_general(xb, kuv_ref[...], _DN_RR,
                                   preferred_element_type=jnp.float32)
    ri = lax.dot_general(xb, kr_ref[...], _DN_RR,
                         preferred_element_type=jnp.float32)
    ri_ref[...] = ri

    @pl.when(pl.program_id(0) == 0)
    def _():
        rs_ref[...] = jnp.zeros_like(rs_ref)

    rs_ref[0:1, :] = rs_ref[0:1, :] + jnp.sum(ri, axis=0, keepdims=True)
    rs_ref[1:2, :] = rs_ref[1:2, :] + jnp.sum(ri * ri, axis=0, keepdims=True)


def _tc_open_body(x_ref, open_ref, x_out_ref):
    x_out_ref[...] = lax.dot_general(x_ref[...], open_ref[...],
                                     (((0,), (1,)), ((), ())),
                                     preferred_element_type=jnp.float32)


def _tc_close_body(x_ref, knc_ref, out_ref):
    out_ref[...] = lax.dot_general(knc_ref[...], x_ref[...],
                                   (((1,), (1,)), ((), ())),
                                   preferred_element_type=jnp.float32)


def _node_block(minor):
    return pl.BlockSpec((NB, minor), lambda i: (i, 0))


def _whole(shape):
    return pl.BlockSpec(shape, lambda i: tuple(0 for _ in shape))


def _tc_open(xn, KNopen):
    return pl.pallas_call(
        _tc_open_body,
        grid=(1,),
        in_specs=[_whole(xn.shape), _whole(KNopen.shape)],
        out_specs=_whole((N_NODES, CH)),
        out_shape=jax.ShapeDtypeStruct((N_NODES, CH), jnp.float32),
    )(xn, KNopen)


def _tc_close(xT, knclose):
    return pl.pallas_call(
        _tc_close_body,
        grid=(1,),
        in_specs=[_whole(xT.shape), _whole(knclose.shape)],
        out_specs=_whole((knclose.shape[0], N_NODES)),
        out_shape=jax.ShapeDtypeStruct((knclose.shape[0], N_NODES),
                                       jnp.float32),
    )(xT, knclose)


def _tc_pre(xT, kuv, kr):
    return pl.pallas_call(
        _tc_pre_body,
        grid=(GRID,),
        in_specs=[
            _node_block(CH),
            _whole(kuv.shape),
            _whole(kr.shape),
        ],
        out_specs=[
            _node_block(2 * CH),
            _node_block(CH),
            _whole((8, CH)),
        ],
        out_shape=[
            jax.ShapeDtypeStruct((N_NODES, 2 * CH), jnp.float32),
            jax.ShapeDtypeStruct((N_NODES, CH), jnp.float32),
            jax.ShapeDtypeStruct((8, CH), jnp.float32),
        ],
    )(xT, kuv, kr)


def _tc_upd_body(x_ref, s0_ref, s1_ref, ri_ref, raff_ref, kn_ref, out_ref):
    r = jnp.maximum(ri_ref[...] * raff_ref[0:1, :] - raff_ref[1:2, :], 0.0)
    jd = lax.dot_general(s0_ref[...], kn_ref[...], _DN_RC,
                         preferred_element_type=jnp.float32)
    out_ref[...] = x_ref[...] - HSTEP * (s1_ref[...] + jd + r)


def _tc_upd(xT, s0, s1, riT, raff, kn):
    return pl.pallas_call(
        _tc_upd_body,
        grid=(GRID,),
        in_specs=[
            _node_block(CH),
            _node_block(CH),
            _node_block(CH),
            _node_block(CH),
            _whole((8, CH)),
            _whole(kn.shape),
        ],
        out_specs=_node_block(CH),
        out_shape=jax.ShapeDtypeStruct((N_NODES, CH), jnp.float32),
    )(xT, s0, s1, riT, raff, kn)


def _edge_aff(stats):
    tot = jnp.sum(stats, axis=0)
    n = jnp.float32(N_EDGES)
    sU = jnp.concatenate([tot[0], tot[1]])
    ssU = jnp.concatenate([tot[2], tot[3]])
    sV = jnp.concatenate([tot[4], tot[5]])
    ssV = jnp.concatenate([tot[6], tot[7]])
    mU = sU / n
    mV = sV / n
    invU = lax.rsqrt(ssU - n * mU * mU + EPS)
    invV = lax.rsqrt(ssV - n * mV * mV + EPS)
    rows = [invU[:16], invU[16:], (mU * invU)[:16], (mU * invU)[16:],
            invV[:16], invV[16:], (mV * invV)[:16], (mV * invV)[16:]]
    return jnp.stack(rows)


def _node_aff(rstats):
    n = jnp.float32(N_NODES)
    m = rstats[0] / n
    inv = lax.rsqrt(rstats[1] - n * m * m + EPS)
    return jnp.concatenate(
        [inv[None], (m * inv)[None], jnp.zeros((6, CH), jnp.float32)])


def kernel(xn, edge_index, KNopen, KNclose, KN, KE, KR):
    ei = edge_index.astype(jnp.int32)
    nL = KN.shape[0]
    xT = _tc_open(xn, KNopen)
    for i in range(nL):
        kuv = jnp.concatenate([KN[i], KE[i]], axis=0)
        yuvT, riT, rstats = _tc_pre(xT, kuv, KR[i])
        aU, aV, stats = _sc_pass_a(yuvT, ei)
        aff = _edge_aff(stats)
        (S,) = _sc_pass_b(aU, aV, ei, aff)
        raff = _node_aff(rstats)
        xT = _tc_upd(xT, S[0], S[1], riT, raff, KN[i])
    return _tc_close(xT, KNclose)


# speedup vs baseline: 1.0153x; 1.0153x over previous
import functools

import jax
import jax.numpy as jnp
from jax import lax
from jax.experimental import pallas as pl
from jax.experimental.pallas import tpu as pltpu
from jax.experimental.pallas import tpu_sc as plsc

N_NODES = 50000
N_EDGES = 800000
CH = 32
LANES = 16
NC = 2
NS = 16
NW = NC * NS
ECH = 128
NCHUNK = N_EDGES // ECH
KA = -(-NCHUNK // NW)
KB = -(-NCHUNK // NS)
RPS = N_NODES // NS
ZR = 125
HSTEP = 0.1
EPS = 0.001

_mesh = plsc.VectorSubcoreMesh(core_axis_name="c", subcore_axis_name="s")
_sc_params = pltpu.CompilerParams(use_tc_tiling_on_sc=False)


@functools.partial(
    pl.kernel,
    mesh=_mesh,
    out_type=[
        jax.ShapeDtypeStruct((N_EDGES, CH), jnp.float32),
        jax.ShapeDtypeStruct((N_EDGES, CH), jnp.float32),
        jax.ShapeDtypeStruct((NW, 8, LANES), jnp.float32),
    ],
    scratch_types=[
        pltpu.VMEM((KA * ECH,), jnp.int32),
        pltpu.VMEM((KA * ECH,), jnp.int32),
        pltpu.VMEM((2, ECH, 2 * CH), jnp.float32),
        pltpu.VMEM((2, ECH, 2 * CH), jnp.float32),
        pltpu.VMEM((2, ECH, CH), jnp.float32),
        pltpu.VMEM((2, ECH, CH), jnp.float32),
        pltpu.VMEM((8, LANES), jnp.float32),
        pltpu.SemaphoreType.DMA,
        pltpu.SemaphoreType.DMA,
        pltpu.SemaphoreType.DMA,
        pltpu.SemaphoreType.DMA,
    ],
    compiler_params=_sc_params,
)
def _sc_pass_a(yuv, ei, aU, aV, stats, idx_i, idx_j, bYi, bYj, wU, wV,
               stats_v, sem0, sem1, wsem0, wsem1):
    cid = lax.axis_index("c")
    sid = lax.axis_index("s")
    wid = sid * NC + cid
    extra = NCHUNK - (KA - 1) * NW
    nk = jnp.where(wid < extra, KA, KA - 1)
    cb = jnp.where(wid < extra, wid * KA, extra + wid * (KA - 1))
    sems = (sem0, sem1)
    wsems = (wsem0, wsem1)
    zero = jnp.zeros((LANES,), jnp.float32)
    for r in range(8):
        stats_v[r, :] = zero

    e0 = cb * ECH
    pltpu.sync_copy(ei.at[0, pl.ds(e0, (KA - 1) * ECH)],
                    idx_i.at[pl.ds(0, (KA - 1) * ECH)])
    pltpu.sync_copy(ei.at[1, pl.ds(e0, (KA - 1) * ECH)],
                    idx_j.at[pl.ds(0, (KA - 1) * ECH)])

    @pl.when(nk == KA)
    def _():
        tail = e0 + (KA - 1) * ECH
        pltpu.sync_copy(ei.at[0, pl.ds(tail, ECH)],
                        idx_i.at[pl.ds((KA - 1) * ECH, ECH)])
        pltpu.sync_copy(ei.at[1, pl.ds(tail, ECH)],
                        idx_j.at[pl.ds((KA - 1) * ECH, ECH)])

    def issue(s, k):
        pltpu.async_copy(yuv.at[idx_i.at[pl.ds(k * ECH, ECH)]], bYi.at[s],
                         sems[s])
        pltpu.async_copy(yuv.at[idx_j.at[pl.ds(k * ECH, ECH)]], bYj.at[s],
                         sems[s])

    def drain_write(s):
        pltpu.make_async_copy(aU.at[pl.ds(0, ECH)], wU.at[s],
                              wsems[s]).wait()
        pltpu.make_async_copy(aU.at[pl.ds(0, ECH)], wV.at[s],
                              wsems[s]).wait()

    def consume(s, k):
        for dst in (bYi.at[s], bYj.at[s]):
            pltpu.make_async_copy(yuv.at[pl.ds(0, ECH)], dst, sems[s]).wait()

        @pl.when(k >= 2)
        def _():
            drain_write(s)

        def row_body(r2, st):
            su0, su1, qu0, qu1, sv0, sv1, qv0, qv1 = st
            for rr in range(4):
                r = 4 * r2 + rr
                s0 = pl.ds(0, LANES)
                s1 = pl.ds(LANES, LANES)
                s2 = pl.ds(2 * LANES, LANES)
                s3 = pl.ds(3 * LANES, LANES)
                au0 = bYi[s, r, s0] - bYj[s, r, s0]
                au1 = bYi[s, r, s1] - bYj[s, r, s1]
                av0 = bYi[s, r, s2] - bYj[s, r, s2]
                av1 = bYi[s, r, s3] - bYj[s, r, s3]
                wU[s, r, pl.ds(0, LANES)] = au0
                wU[s, r, pl.ds(LANES, LANES)] = au1
                wV[s, r, pl.ds(0, LANES)] = av0
                wV[s, r, pl.ds(LANES, LANES)] = av1
                su0 = su0 + au0
                su1 = su1 + au1
                qu0 = qu0 + au0 * au0
                qu1 = qu1 + au1 * au1
                sv0 = sv0 + av0
                sv1 = sv1 + av1
                qv0 = qv0 + av0 * av0
                qv1 = qv1 + av1 * av1
            return (su0, su1, qu0, qu1, sv0, sv1, qv0, qv1)

        st = lax.fori_loop(0, ECH // 4, row_body, (zero,) * 8)
        for r in range(8):
            stats_v[r, :] = stats_v[r, :] + st[r]
        base = (cb + k) * ECH
        pltpu.async_copy(wU.at[s], aU.at[pl.ds(base, ECH)], wsems[s])
        pltpu.async_copy(wV.at[s], aV.at[pl.ds(base, ECH)], wsems[s])

    issue(0, 0)

    def pair_body(t, carry):
        k0 = 2 * t
        k1 = k0 + 1

        @pl.when(k1 < nk)
        def _():
            issue(1, k1)

        consume(0, k0)

        @pl.when(k0 + 2 < nk)
        def _():
            issue(0, k0 + 2)

        @pl.when(k1 < nk)
        def _():
            consume(1, k1)

        return carry

    lax.fori_loop(0, KA // 2, pair_body, 0)
    drain_write(0)
    drain_write(1)
    pltpu.sync_copy(stats_v, stats.at[wid])


@functools.partial(
    pl.kernel,
    mesh=_mesh,
    out_type=[
        jax.ShapeDtypeStruct((NC, N_NODES, CH), jnp.float32),
    ],
    scratch_types=[
        pltpu.VMEM_SHARED((N_NODES, CH), jnp.float32),
        pltpu.VMEM((2, ECH, CH), jnp.float32),
        pltpu.VMEM((2, ECH, CH), jnp.float32),
        pltpu.VMEM((2, ECH, CH), jnp.float32),
        pltpu.VMEM((2, ECH), jnp.int32),
        pltpu.VMEM((2, ECH), jnp.int32),
        pltpu.VMEM((ZR, CH), jnp.float32),
        pltpu.VMEM((8, LANES), jnp.float32),
        pltpu.SemaphoreType.DMA,
        pltpu.SemaphoreType.DMA,
        pltpu.SemaphoreType.DMA,
        pltpu.SemaphoreType.DMA,
    ],
    compiler_params=_sc_params,
)
def _sc_pass_b(aU, aV, ei, aff, S, acc, aBuf, sBuf, nBuf, idx_i, idx_j, zbuf,
               aff_v, sem0, sem1, ssem0, ssem1):
    cid = lax.axis_index("c")
    sid = lax.axis_index("s")
    sems = (sem0, sem1)
    ssems = (ssem0, ssem1)
    extra = NCHUNK - (KB - 1) * NS
    nk = jnp.where(sid < extra, KB, KB - 1)
    zero = jnp.zeros((LANES,), jnp.float32)

    def zrow(r, carry):
        zbuf[r, pl.ds(0, LANES)] = zero
        zbuf[r, pl.ds(LANES, LANES)] = zero
        return carry

    lax.fori_loop(0, ZR, zrow, 0)
    for t in range(RPS // ZR):
        pltpu.sync_copy(zbuf, acc.at[pl.ds(sid * RPS + t * ZR, ZR)])
    plsc.subcore_barrier()

    pltpu.sync_copy(aff, aff_v)
    mul0 = aff_v[4 * cid + 0, :]
    mul1 = aff_v[4 * cid + 1, :]
    sub0 = aff_v[4 * cid + 2, :]
    sub1 = aff_v[4 * cid + 3, :]
    outscale = jnp.where(cid == 0, 1.0, 0.5).astype(jnp.float32)
    sign = jnp.where(cid == 0, -1.0, 1.0).astype(jnp.float32)

    def issue(s, k):
        base = (k * NS + sid) * ECH

        @pl.when(cid == 0)
        def _():
            pltpu.async_copy(aU.at[pl.ds(base, ECH)], aBuf.at[s], sems[s])

        @pl.when(cid == 1)
        def _():
            pltpu.async_copy(aV.at[pl.ds(base, ECH)], aBuf.at[s], sems[s])

        pltpu.async_copy(ei.at[0, pl.ds(base, ECH)], idx_i.at[s], sems[s])
        pltpu.async_copy(ei.at[1, pl.ds(base, ECH)], idx_j.at[s], sems[s])

    def drain_scatter(s):
        pltpu.make_async_copy(aU.at[pl.ds(0, ECH)], sBuf.at[s],
                              ssems[s]).wait()
        pltpu.make_async_copy(aU.at[pl.ds(0, ECH)], nBuf.at[s],
                              ssems[s]).wait()

    def consume(s, k):
        pltpu.make_async_copy(aU.at[pl.ds(0, ECH)], aBuf.at[s],
                              sems[s]).wait()
        pltpu.make_async_copy(ei.at[0, pl.ds(0, ECH)], idx_i.at[s],
                              sems[s]).wait()
        pltpu.make_async_copy(ei.at[0, pl.ds(0, ECH)], idx_j.at[s],
                              sems[s]).wait()

        @pl.when(k >= 2)
        def _():
            drain_scatter(s)

        def row_body(r2, rc):
            for rr in range(2):
                r = 2 * r2 + rr
                for h, (m, sb) in enumerate(((mul0, sub0), (mul1, sub1))):
                    sl = pl.ds(h * LANES, LANES)
                    a = aBuf[s, r, sl]
                    d = jnp.maximum(a * m - sb, 0.0) * outscale
                    sBuf[s, r, sl] = d
                    nBuf[s, r, sl] = d * sign
            return rc

        lax.fori_loop(0, ECH // 2, row_body, 0)
        pltpu.async_copy(sBuf.at[s], acc.at[idx_i.at[s]], ssems[s], add=True)
        pltpu.async_copy(nBuf.at[s], acc.at[idx_j.at[s]], ssems[s], add=True)

    issue(0, 0)

    def pair_body(t, carry):
        k0 = 2 * t
        k1 = k0 + 1

        @pl.when(k1 < nk)
        def _():
            issue(1, k1)

        @pl.when(k0 < nk)
        def _():
            consume(0, k0)

        @pl.when(k0 + 2 < nk)
        def _():
            issue(0, k0 + 2)

        @pl.when(k1 < nk)
        def _():
            consume(1, k1)

        return carry

    lax.fori_loop(0, (KB + 1) // 2, pair_body, 0)
    drain_scatter(0)
    drain_scatter(1)
    plsc.subcore_barrier()
    for t in range(RPS // ZR):
        row0 = sid * RPS + t * ZR
        pltpu.sync_copy(acc.at[pl.ds(row0, ZR)], S.at[cid, pl.ds(row0, ZR)])


NB = 5000
GRID = N_NODES // NB
_DN_RR = (((1,), (1,)), ((), ()))
_DN_RC = (((1,), (0,)), ((), ()))


def _tc_pre_body(x_ref, kuv_ref, kr_ref, yuv_ref, ri_ref, rs_ref):
    xb = x_ref[...]
    yuv_ref[...] = lax.dot_general(xb, kuv_ref[...], _DN_RR,
                                   preferred_element_type=jnp.float32)
    ri = lax.dot_general(xb, kr_ref[...], _DN_RR,
                         preferred_element_type=jnp.float32)
    ri_ref[...] = ri

    @pl.when(pl.program_id(0) == 0)
    def _():
        rs_ref[...] = jnp.zeros_like(rs_ref)

    rs_ref[0:1, :] = rs_ref[0:1, :] + jnp.sum(ri, axis=0, keepdims=True)
    rs_ref[1:2, :] = rs_ref[1:2, :] + jnp.sum(ri * ri, axis=0, keepdims=True)


def _tc_open_body(x_ref, open_ref, x_out_ref):
    x_out_ref[...] = lax.dot_general(x_ref[...], open_ref[...], _DN_RR,
                                     preferred_element_type=jnp.float32)


def _tc_close_body(x_ref, knc_ref, out_ref):
    out_ref[...] = lax.dot_general(knc_ref[...], x_ref[...],
                                   (((1,), (1,)), ((), ())),
                                   preferred_element_type=jnp.float32)


def _node_block(minor):
    return pl.BlockSpec((NB, minor), lambda i: (i, 0))


def _whole(shape):
    return pl.BlockSpec(shape, lambda i: tuple(0 for _ in shape))


def _tc_open(xnT, KNopen):
    return pl.pallas_call(
        _tc_open_body,
        grid=(GRID,),
        in_specs=[_node_block(xnT.shape[1]), _whole(KNopen.shape)],
        out_specs=_node_block(CH),
        out_shape=jax.ShapeDtypeStruct((N_NODES, CH), jnp.float32),
    )(xnT, KNopen)


def _tc_close(xT, knclose):
    return pl.pallas_call(
        _tc_close_body,
        grid=(1,),
        in_specs=[_whole(xT.shape), _whole(knclose.shape)],
        out_specs=_whole((knclose.shape[0], N_NODES)),
        out_shape=jax.ShapeDtypeStruct((knclose.shape[0], N_NODES),
                                       jnp.float32),
    )(xT, knclose)


def _tc_pre(xT, kuv, kr):
    return pl.pallas_call(
        _tc_pre_body,
        grid=(GRID,),
        in_specs=[
            _node_block(CH),
            _whole(kuv.shape),
            _whole(kr.shape),
        ],
        out_specs=[
            _node_block(2 * CH),
            _node_block(CH),
            _whole((8, CH)),
        ],
        out_shape=[
            jax.ShapeDtypeStruct((N_NODES, 2 * CH), jnp.float32),
            jax.ShapeDtypeStruct((N_NODES, CH), jnp.float32),
            jax.ShapeDtypeStruct((8, CH), jnp.float32),
        ],
    )(xT, kuv, kr)


def _tc_upd_body(x_ref, s0_ref, s1_ref, ri_ref, raff_ref, kn_ref, out_ref):
    r = jnp.maximum(ri_ref[...] * raff_ref[0:1, :] - raff_ref[1:2, :], 0.0)
    jd = lax.dot_general(s0_ref[...], kn_ref[...], _DN_RC,
                         preferred_element_type=jnp.float32)
    out_ref[...] = x_ref[...] - HSTEP * (s1_ref[...] + jd + r)


def _tc_upd(xT, s0, s1, riT, raff, kn):
    return pl.pallas_call(
        _tc_upd_body,
        grid=(GRID,),
        in_specs=[
            _node_block(CH),
            _node_block(CH),
            _node_block(CH),
            _node_block(CH),
            _whole((8, CH)),
            _whole(kn.shape),
        ],
        out_specs=_node_block(CH),
        out_shape=jax.ShapeDtypeStruct((N_NODES, CH), jnp.float32),
    )(xT, s0, s1, riT, raff, kn)


def _edge_aff(stats):
    tot = jnp.sum(stats, axis=0)
    n = jnp.float32(N_EDGES)
    sU = jnp.concatenate([tot[0], tot[1]])
    ssU = jnp.concatenate([tot[2], tot[3]])
    sV = jnp.concatenate([tot[4], tot[5]])
    ssV = jnp.concatenate([tot[6], tot[7]])
    mU = sU / n
    mV = sV / n
    invU = lax.rsqrt(ssU - n * mU * mU + EPS)
    invV = lax.rsqrt(ssV - n * mV * mV + EPS)
    rows = [invU[:16], invU[16:], (mU * invU)[:16], (mU * invU)[16:],
            invV[:16], invV[16:], (mV * invV)[:16], (mV * invV)[16:]]
    return jnp.stack(rows)


def _node_aff(rstats):
    n = jnp.float32(N_NODES)
    m = rstats[0] / n
    inv = lax.rsqrt(rstats[1] - n * m * m + EPS)
    return jnp.concatenate(
        [inv[None], (m * inv)[None], jnp.zeros((6, CH), jnp.float32)])


def kernel(xn, edge_index, KNopen, KNclose, KN, KE, KR):
    ei = edge_index.astype(jnp.int32)
    nL = KN.shape[0]
    xT = _tc_open(xn.T, KNopen)
    for i in range(nL):
        kuv = jnp.concatenate([KN[i], KE[i]], axis=0)
        yuvT, riT, rstats = _tc_pre(xT, kuv, KR[i])
        aU, aV, stats = _sc_pass_a(yuvT, ei)
        aff = _edge_aff(stats)
        (S,) = _sc_pass_b(aU, aV, ei, aff)
        raff = _node_aff(rstats)
        xT = _tc_upd(xT, S[0], S[1], riT, raff, KN[i])
    return _tc_close(xT, KNclose)


# speedup vs baseline: 1.0272x; 1.0117x over previous
import functools

import jax
import jax.numpy as jnp
from jax import lax
from jax.experimental import pallas as pl
from jax.experimental.pallas import tpu as pltpu
from jax.experimental.pallas import tpu_sc as plsc

N_NODES = 50000
N_EDGES = 800000
CH = 32
LANES = 16
NC = 2
NS = 16
NW = NC * NS
ECH = 128
NCHUNK = N_EDGES // ECH
KA = -(-NCHUNK // NW)
KB = -(-NCHUNK // NS)
RPS = N_NODES // NS
ZR = 125
HSTEP = 0.1
EPS = 0.001

_mesh = plsc.VectorSubcoreMesh(core_axis_name="c", subcore_axis_name="s")
_sc_params = pltpu.CompilerParams(use_tc_tiling_on_sc=False)


@functools.partial(
    pl.kernel,
    mesh=_mesh,
    out_type=[
        jax.ShapeDtypeStruct((N_EDGES, CH), jnp.float32),
        jax.ShapeDtypeStruct((N_EDGES, CH), jnp.float32),
        jax.ShapeDtypeStruct((NW, 8, LANES), jnp.float32),
    ],
    scratch_types=[
        pltpu.VMEM((KA * ECH,), jnp.int32),
        pltpu.VMEM((KA * ECH,), jnp.int32),
        pltpu.VMEM((2, ECH, 2 * CH), jnp.float32),
        pltpu.VMEM((2, ECH, 2 * CH), jnp.float32),
        pltpu.VMEM((2, ECH, CH), jnp.float32),
        pltpu.VMEM((2, ECH, CH), jnp.float32),
        pltpu.VMEM((8, LANES), jnp.float32),
        pltpu.SemaphoreType.DMA,
        pltpu.SemaphoreType.DMA,
        pltpu.SemaphoreType.DMA,
        pltpu.SemaphoreType.DMA,
    ],
    compiler_params=_sc_params,
)
def _sc_pass_a(yuv, ei, aU, aV, stats, idx_i, idx_j, bYi, bYj, wU, wV,
               stats_v, sem0, sem1, wsem0, wsem1):
    cid = lax.axis_index("c")
    sid = lax.axis_index("s")
    wid = sid * NC + cid
    extra = NCHUNK - (KA - 1) * NW
    nk = jnp.where(wid < extra, KA, KA - 1)
    cb = jnp.where(wid < extra, wid * KA, extra + wid * (KA - 1))
    sems = (sem0, sem1)
    wsems = (wsem0, wsem1)
    zero = jnp.zeros((LANES,), jnp.float32)
    for r in range(8):
        stats_v[r, :] = zero

    e0 = cb * ECH
    pltpu.sync_copy(ei.at[0, pl.ds(e0, (KA - 1) * ECH)],
                    idx_i.at[pl.ds(0, (KA - 1) * ECH)])
    pltpu.sync_copy(ei.at[1, pl.ds(e0, (KA - 1) * ECH)],
                    idx_j.at[pl.ds(0, (KA - 1) * ECH)])

    @pl.when(nk == KA)
    def _():
        tail = e0 + (KA - 1) * ECH
        pltpu.sync_copy(ei.at[0, pl.ds(tail, ECH)],
                        idx_i.at[pl.ds((KA - 1) * ECH, ECH)])
        pltpu.sync_copy(ei.at[1, pl.ds(tail, ECH)],
                        idx_j.at[pl.ds((KA - 1) * ECH, ECH)])

    def issue(s, k):
        pltpu.async_copy(yuv.at[idx_i.at[pl.ds(k * ECH, ECH)]], bYi.at[s],
                         sems[s])
        pltpu.async_copy(yuv.at[idx_j.at[pl.ds(k * ECH, ECH)]], bYj.at[s],
                         sems[s])

    def drain_write(s):
        pltpu.make_async_copy(aU.at[pl.ds(0, ECH)], wU.at[s],
                              wsems[s]).wait()
        pltpu.make_async_copy(aU.at[pl.ds(0, ECH)], wV.at[s],
                              wsems[s]).wait()

    def consume(s, k):
        for dst in (bYi.at[s], bYj.at[s]):
            pltpu.make_async_copy(yuv.at[pl.ds(0, ECH)], dst, sems[s]).wait()

        @pl.when(k >= 2)
        def _():
            drain_write(s)

        def row_body(r2, st):
            su0, su1, qu0, qu1, sv0, sv1, qv0, qv1 = st
            for rr in range(4):
                r = 4 * r2 + rr
                s0 = pl.ds(0, LANES)
                s1 = pl.ds(LANES, LANES)
                s2 = pl.ds(2 * LANES, LANES)
                s3 = pl.ds(3 * LANES, LANES)
                au0 = bYi[s, r, s0] - bYj[s, r, s0]
                au1 = bYi[s, r, s1] - bYj[s, r, s1]
                av0 = bYi[s, r, s2] - bYj[s, r, s2]
                av1 = bYi[s, r, s3] - bYj[s, r, s3]
                wU[s, r, pl.ds(0, LANES)] = au0
                wU[s, r, pl.ds(LANES, LANES)] = au1
                wV[s, r, pl.ds(0, LANES)] = av0
                wV[s, r, pl.ds(LANES, LANES)] = av1
                su0 = su0 + au0
                su1 = su1 + au1
                qu0 = qu0 + au0 * au0
                qu1 = qu1 + au1 * au1
                sv0 = sv0 + av0
                sv1 = sv1 + av1
                qv0 = qv0 + av0 * av0
                qv1 = qv1 + av1 * av1
            return (su0, su1, qu0, qu1, sv0, sv1, qv0, qv1)

        st = lax.fori_loop(0, ECH // 4, row_body, (zero,) * 8)
        for r in range(8):
            stats_v[r, :] = stats_v[r, :] + st[r]
        base = (cb + k) * ECH
        pltpu.async_copy(wU.at[s], aU.at[pl.ds(base, ECH)], wsems[s])
        pltpu.async_copy(wV.at[s], aV.at[pl.ds(base, ECH)], wsems[s])

    issue(0, 0)

    def pair_body(t, carry):
        k0 = 2 * t
        k1 = k0 + 1

        @pl.when(k1 < nk)
        def _():
            issue(1, k1)

        consume(0, k0)

        @pl.when(k0 + 2 < nk)
        def _():
            issue(0, k0 + 2)

        @pl.when(k1 < nk)
        def _():
            consume(1, k1)

        return carry

    lax.fori_loop(0, KA // 2, pair_body, 0)
    drain_write(0)
    drain_write(1)
    pltpu.sync_copy(stats_v, stats.at[wid])


@functools.partial(
    pl.kernel,
    mesh=_mesh,
    out_type=[
        jax.ShapeDtypeStruct((NC, N_NODES, CH), jnp.float32),
    ],
    scratch_types=[
        pltpu.VMEM_SHARED((N_NODES, CH), jnp.float32),
        pltpu.VMEM((2, ECH, CH), jnp.float32),
        pltpu.VMEM((2, ECH, CH), jnp.float32),
        pltpu.VMEM((2, ECH, CH), jnp.float32),
        pltpu.VMEM((2, ECH), jnp.int32),
        pltpu.VMEM((2, ECH), jnp.int32),
        pltpu.VMEM((ZR, CH), jnp.float32),
        pltpu.VMEM((8, LANES), jnp.float32),
        pltpu.SemaphoreType.DMA,
        pltpu.SemaphoreType.DMA,
        pltpu.SemaphoreType.DMA,
        pltpu.SemaphoreType.DMA,
    ],
    compiler_params=_sc_params,
)
def _sc_pass_b(aU, aV, ei, aff, S, acc, aBuf, sBuf, nBuf, idx_i, idx_j, zbuf,
               aff_v, sem0, sem1, ssem0, ssem1):
    cid = lax.axis_index("c")
    sid = lax.axis_index("s")
    sems = (sem0, sem1)
    ssems = (ssem0, ssem1)
    extra = NCHUNK - (KB - 1) * NS
    nk = jnp.where(sid < extra, KB, KB - 1)
    zero = jnp.zeros((LANES,), jnp.float32)

    def zrow(r, carry):
        zbuf[r, pl.ds(0, LANES)] = zero
        zbuf[r, pl.ds(LANES, LANES)] = zero
        return carry

    lax.fori_loop(0, ZR, zrow, 0)
    for t in range(RPS // ZR):
        pltpu.sync_copy(zbuf, acc.at[pl.ds(sid * RPS + t * ZR, ZR)])
    plsc.subcore_barrier()

    pltpu.sync_copy(aff, aff_v)
    mul0 = aff_v[4 * cid + 0, :]
    mul1 = aff_v[4 * cid + 1, :]
    sub0 = aff_v[4 * cid + 2, :]
    sub1 = aff_v[4 * cid + 3, :]
    outscale = jnp.where(cid == 0, 1.0, 0.5).astype(jnp.float32)
    sign = jnp.where(cid == 0, -1.0, 1.0).astype(jnp.float32)

    def issue(s, k):
        base = (k * NS + sid) * ECH

        @pl.when(cid == 0)
        def _():
            pltpu.async_copy(aU.at[pl.ds(base, ECH)], aBuf.at[s], sems[s])

        @pl.when(cid == 1)
        def _():
            pltpu.async_copy(aV.at[pl.ds(base, ECH)], aBuf.at[s], sems[s])

        pltpu.async_copy(ei.at[0, pl.ds(base, ECH)], idx_i.at[s], sems[s])
        pltpu.async_copy(ei.at[1, pl.ds(base, ECH)], idx_j.at[s], sems[s])

    def drain_scatter(s):
        pltpu.make_async_copy(aU.at[pl.ds(0, ECH)], sBuf.at[s],
                              ssems[s]).wait()
        pltpu.make_async_copy(aU.at[pl.ds(0, ECH)], nBuf.at[s],
                              ssems[s]).wait()

    def consume(s, k):
        pltpu.make_async_copy(aU.at[pl.ds(0, ECH)], aBuf.at[s],
                              sems[s]).wait()
        pltpu.make_async_copy(ei.at[0, pl.ds(0, ECH)], idx_i.at[s],
                              sems[s]).wait()
        pltpu.make_async_copy(ei.at[0, pl.ds(0, ECH)], idx_j.at[s],
                              sems[s]).wait()

        @pl.when(k >= 2)
        def _():
            drain_scatter(s)

        def row_body(r2, rc):
            for rr in range(2):
                r = 2 * r2 + rr
                for h, (m, sb) in enumerate(((mul0, sub0), (mul1, sub1))):
                    sl = pl.ds(h * LANES, LANES)
                    a = aBuf[s, r, sl]
                    d = jnp.maximum(a * m - sb, 0.0) * outscale
                    sBuf[s, r, sl] = d
                    nBuf[s, r, sl] = d * sign
            return rc

        lax.fori_loop(0, ECH // 2, row_body, 0)
        pltpu.async_copy(sBuf.at[s], acc.at[idx_i.at[s]], ssems[s], add=True)
        pltpu.async_copy(nBuf.at[s], acc.at[idx_j.at[s]], ssems[s], add=True)

    issue(0, 0)

    def pair_body(t, carry):
        k0 = 2 * t
        k1 = k0 + 1

        @pl.when(k1 < nk)
        def _():
            issue(1, k1)

        @pl.when(k0 < nk)
        def _():
            consume(0, k0)

        @pl.when(k0 + 2 < nk)
        def _():
            issue(0, k0 + 2)

        @pl.when(k1 < nk)
        def _():
            consume(1, k1)

        return carry

    lax.fori_loop(0, (KB + 1) // 2, pair_body, 0)
    drain_scatter(0)
    drain_scatter(1)
    plsc.subcore_barrier()
    row0 = sid * RPS
    pltpu.sync_copy(acc.at[pl.ds(row0, RPS)], S.at[cid, pl.ds(row0, RPS)])


NB = 5000
GRID = N_NODES // NB
_DN_RR = (((1,), (1,)), ((), ()))
_DN_RC = (((1,), (0,)), ((), ()))


def _tc_pre_body(x_ref, kuv_ref, kr_ref, yuv_ref, ri_ref, rs_ref):
    xb = x_ref[...]
    yuv_ref[...] = lax.dot_general(xb, kuv_ref[...], _DN_RR,
                                   preferred_element_type=jnp.float32)
    ri = lax.dot_general(xb, kr_ref[...], _DN_RR,
                         preferred_element_type=jnp.float32)
    ri_ref[...] = ri

    @pl.when(pl.program_id(0) == 0)
    def _():
        rs_ref[...] = jnp.zeros_like(rs_ref)

    rs_ref[0:1, :] = rs_ref[0:1, :] + jnp.sum(ri, axis=0, keepdims=True)
    rs_ref[1:2, :] = rs_ref[1:2, :] + jnp.sum(ri * ri, axis=0, keepdims=True)


def _tc_open_body(x_ref, open_ref, x_out_ref):
    x_out_ref[...] = lax.dot_general(x_ref[...], open_ref[...], _DN_RR,
                                     preferred_element_type=jnp.float32)


def _tc_close_body(x_ref, knc_ref, out_ref):
    out_ref[...] = lax.dot_general(knc_ref[...], x_ref[...],
                                   (((1,), (1,)), ((), ())),
                                   preferred_element_type=jnp.float32)


def _node_block(minor):
    return pl.BlockSpec((NB, minor), lambda i: (i, 0))


def _whole(shape):
    return pl.BlockSpec(shape, lambda i: tuple(0 for _ in shape))


def _tc_open(xnT, KNopen):
    return pl.pallas_call(
        _tc_open_body,
        grid=(GRID,),
        in_specs=[_node_block(xnT.shape[1]), _whole(KNopen.shape)],
        out_specs=_node_block(CH),
        out_shape=jax.ShapeDtypeStruct((N_NODES, CH), jnp.float32),
    )(xnT, KNopen)


def _tc_close(xT, knclose):
    return pl.pallas_call(
        _tc_close_body,
        grid=(1,),
        in_specs=[_whole(xT.shape), _whole(knclose.shape)],
        out_specs=_whole((knclose.shape[0], N_NODES)),
        out_shape=jax.ShapeDtypeStruct((knclose.shape[0], N_NODES),
                                       jnp.float32),
    )(xT, knclose)


def _tc_pre(xT, kuv, kr):
    return pl.pallas_call(
        _tc_pre_body,
        grid=(GRID,),
        in_specs=[
            _node_block(CH),
            _whole(kuv.shape),
            _whole(kr.shape),
        ],
        out_specs=[
            _node_block(2 * CH),
            _node_block(CH),
            _whole((8, CH)),
        ],
        out_shape=[
            jax.ShapeDtypeStruct((N_NODES, 2 * CH), jnp.float32),
            jax.ShapeDtypeStruct((N_NODES, CH), jnp.float32),
            jax.ShapeDtypeStruct((8, CH), jnp.float32),
        ],
    )(xT, kuv, kr)


def _tc_upd_body(x_ref, s0_ref, s1_ref, ri_ref, raff_ref, kn_ref, out_ref):
    r = jnp.maximum(ri_ref[...] * raff_ref[0:1, :] - raff_ref[1:2, :], 0.0)
    jd = lax.dot_general(s0_ref[...], kn_ref[...], _DN_RC,
                         preferred_element_type=jnp.float32)
    out_ref[...] = x_ref[...] - HSTEP * (s1_ref[...] + jd + r)


def _tc_upd(xT, s0, s1, riT, raff, kn):
    return pl.pallas_call(
        _tc_upd_body,
        grid=(GRID,),
        in_specs=[
            _node_block(CH),
            _node_block(CH),
            _node_block(CH),
            _node_block(CH),
            _whole((8, CH)),
            _whole(kn.shape),
        ],
        out_specs=_node_block(CH),
        out_shape=jax.ShapeDtypeStruct((N_NODES, CH), jnp.float32),
    )(xT, s0, s1, riT, raff, kn)


def _edge_aff(stats):
    tot = jnp.sum(stats, axis=0)
    n = jnp.float32(N_EDGES)
    sU = jnp.concatenate([tot[0], tot[1]])
    ssU = jnp.concatenate([tot[2], tot[3]])
    sV = jnp.concatenate([tot[4], tot[5]])
    ssV = jnp.concatenate([tot[6], tot[7]])
    mU = sU / n
    mV = sV / n
    invU = lax.rsqrt(ssU - n * mU * mU + EPS)
    invV = lax.rsqrt(ssV - n * mV * mV + EPS)
    rows = [invU[:16], invU[16:], (mU * invU)[:16], (mU * invU)[16:],
            invV[:16], invV[16:], (mV * invV)[:16], (mV * invV)[16:]]
    return jnp.stack(rows)


def _node_aff(rstats):
    n = jnp.float32(N_NODES)
    m = rstats[0] / n
    inv = lax.rsqrt(rstats[1] - n * m * m + EPS)
    return jnp.concatenate(
        [inv[None], (m * inv)[None], jnp.zeros((6, CH), jnp.float32)])


def kernel(xn, edge_index, KNopen, KNclose, KN, KE, KR):
    ei = edge_index.astype(jnp.int32)
    nL = KN.shape[0]
    xT = _tc_open(xn.T, KNopen)
    for i in range(nL):
        kuv = jnp.concatenate([KN[i], KE[i]], axis=0)
        yuvT, riT, rstats = _tc_pre(xT, kuv, KR[i])
        aU, aV, stats = _sc_pass_a(yuvT, ei)
        aff = _edge_aff(stats)
        (S,) = _sc_pass_b(aU, aV, ei, aff)
        raff = _node_aff(rstats)
        xT = _tc_upd(xT, S[0], S[1], riT, raff, KN[i])
    return _tc_close(xT, KNclose)
